# baseline (speedup 1.0000x reference)
import functools
import math

import jax
import jax.numpy as jnp
from jax import lax
from jax.experimental import pallas as pl
from jax.experimental.pallas import tpu as pltpu
from jax.experimental.pallas import tpu_sc as plsc

N = 10000
NP = 10240
E = 320000
D = 128
H = 128
L = 64

NC = 2
NS = 16
NW = NC * NS
WK = 128
WPER = 80
NWIN = NW * WPER
EP = NWIN * WK
ROWS_T = NP // NS

def _win_perm():
    import numpy as np
    perm = np.empty((NW, WPER), dtype=np.int32)
    r = 0
    d = E // WK
    for w in range(NW):
        nreal = 78 if w < 28 else 79
        perm[w, :nreal] = np.arange(r, r + nreal)
        perm[w, nreal:] = np.arange(d, d + WPER - nreal)
        r += nreal
        d += WPER - nreal
    return perm.reshape(-1)


_WIN_PERM = _win_perm()

_mesh = functools.partial(
    plsc.VectorSubcoreMesh,
    core_axis_name="c", subcore_axis_name="s",
    num_cores=NC, num_subcores=NS,
)


def _worker_id():
    c = lax.axis_index("c")
    s = lax.axis_index("s")
    return s * NC + c, c, s


def _deg_body(dst2_hbm, zr_hbm, out_hbm, dstb, ones_v, accd):
    wid, c, s = _worker_id()
    for i in range(WK // 16):
        ones_v[pl.ds(i * 16, 16)] = jnp.full((16,), 1.0, jnp.float32)
    pltpu.sync_copy(zr_hbm, accd.at[pl.ds(s * ROWS_T, ROWS_T)])
    pltpu.sync_copy(dst2_hbm.at[pl.ds(wid * WPER, WPER)], dstb)
    plsc.subcore_barrier()

    def body(j, carry):
        pltpu.sync_copy(ones_v, accd.at[dstb.at[j]], add=True)
        return carry

    lax.fori_loop(0, WPER, body, 0)

    plsc.subcore_barrier()
    pltpu.sync_copy(accd.at[pl.ds(s * ROWS_T, ROWS_T)],
                    out_hbm.at[pl.ds(c * NP + s * ROWS_T, ROWS_T)])


def _sc_degrees(dst2, zr1d):
    fn = pl.kernel(
        _deg_body,
        out_type=jax.ShapeDtypeStruct((NC * NP,), jnp.float32),
        mesh=_mesh(),
        scratch_types=[
            pltpu.VMEM((WPER, WK), jnp.int32),
            pltpu.VMEM((WK,), jnp.float32),
            pltpu.VMEM_SHARED((NP,), jnp.float32),
        ],
    )
    return fn(dst2, zr1d)


NBUF = 2
WSEG = 16


def _edge_body(u_hbm, src2_hbm, dst2_hbm, zr_hbm, out_hbm,
               srcb, dstb, rows, acc, g0, g1, s0, s1):
    wid, c, s = _worker_id()
    gsems = (g0, g1)
    ssems = (s0, s1)
    base = wid * WPER
    pltpu.sync_copy(zr_hbm, acc.at[pl.ds(s * ROWS_T, ROWS_T)])
    plsc.subcore_barrier()

    HK = WK // 2

    def gstart(j, b):
        pltpu.async_copy(u_hbm.at[srcb.at[j, pl.ds(0, HK)]],
                         rows.at[b, pl.ds(0, HK)], gsems[b])
        return pltpu.async_copy(u_hbm.at[srcb.at[j, pl.ds(HK, HK)]],
                                rows.at[b, pl.ds(HK, HK)], gsems[b])

    def sstart(j, b):
        return pltpu.async_copy(rows.at[b], acc.at[dstb.at[j]],
                                ssems[b], add=True)

    for seg in range(WPER // WSEG):
        pltpu.sync_copy(src2_hbm.at[pl.ds(base + seg * WSEG, WSEG)], srcb)
        pltpu.sync_copy(dst2_hbm.at[pl.ds(base + seg * WSEG, WSEG)], dstb)
        gh = [gstart(0, 0), None]
        sh = [None, None]
        for j in range(WSEG):
            b = j % NBUF
            ob = 1 - b
            gh[b].wait()
            gh[b].wait()
            sh[b] = sstart(j, b)
            if sh[ob] is not None:
                sh[ob].wait()
            if j + 1 < WSEG:
                gh[ob] = gstart(j + 1, ob)
        sh[(WSEG - 1) % NBUF].wait()

    plsc.subcore_barrier()
    pltpu.sync_copy(acc.at[pl.ds(s * ROWS_T, ROWS_T)],
                    out_hbm.at[c, pl.ds(s * ROWS_T, ROWS_T)])


def _sc_edge_aggregate(u, src2, dst2, zrows):
    fn = pl.kernel(
        _edge_body,
        out_type=jax.ShapeDtypeStruct((NC, NP, H), jnp.float32),
        mesh=_mesh(),
        scratch_types=[
            pltpu.VMEM((WSEG, WK), jnp.int32),
            pltpu.VMEM((WSEG, WK), jnp.int32),
            pltpu.VMEM((NBUF, WK, H), jnp.float32),
            pltpu.VMEM_SHARED((NP, H), jnp.float32),
            pltpu.SemaphoreType.DMA,
            pltpu.SemaphoreType.DMA,
            pltpu.SemaphoreType.DMA,
            pltpu.SemaphoreType.DMA,
        ],
        name="edge_aggregate",
    )
    return fn(u, src2, dst2, zrows)


BN = 2048


def _dinv_of(degp_ref):
    deg = degp_ref[:, 0] + degp_ref[:, 1] + 1.0
    return lax.rsqrt(deg)


def _mm1_body(x_ref, w_ref, degp_ref, o_ref):
    dinv = _dinv_of(degp_ref)
    hw = jnp.dot(x_ref[...], w_ref[...],
                 preferred_element_type=jnp.float32)
    o_ref[...] = hw * dinv[:, None]


def _tc_mm1(x, W1, degp):
    return pl.pallas_call(
        _mm1_body,
        grid=(NP // BN,),
        in_specs=[
            pl.BlockSpec((BN, D), lambda i: (i, 0)),
            pl.BlockSpec((D, H), lambda i: (0, 0)),
            pl.BlockSpec((BN, NC), lambda i: (i, 0)),
        ],
        out_specs=pl.BlockSpec((BN, H), lambda i: (i, 0)),
        out_shape=jax.ShapeDtypeStruct((NP, H), jnp.float32),
    )(x, W1, degp)


def _mm2_body(p_ref, u_ref, degp_ref, b1_ref, w2_ref, o_ref):
    dinv = _dinv_of(degp_ref)
    agg = (p_ref[0] + p_ref[1] + u_ref[...]) * dinv[:, None] + b1_ref[...]
    h1 = jnp.maximum(agg, 0.0)
    hw = jnp.dot(h1, w2_ref[...], preferred_element_type=jnp.float32)
    o_ref[...] = hw * dinv[:, None]


def _tc_mm2(P1, u1, degp, b1, W2):
    return pl.pallas_call(
        _mm2_body,
        grid=(NP // BN,),
        in_specs=[
            pl.BlockSpec((NC, BN, H), lambda i: (0, i, 0)),
            pl.BlockSpec((BN, H), lambda i: (i, 0)),
            pl.BlockSpec((BN, NC), lambda i: (i, 0)),
            pl.BlockSpec((1, H), lambda i: (0, 0)),
            pl.BlockSpec((H, H), lambda i: (0, 0)),
        ],
        out_specs=pl.BlockSpec((BN, H), lambda i: (i, 0)),
        out_shape=jax.ShapeDtypeStruct((NP, H), jnp.float32),
    )(P1, u1, degp, b1, W2)


_SQRT2 = math.sqrt(2.0)


def _gelu(x):
    return 0.5 * x * (1.0 + lax.erf(x / _SQRT2))


def _mm3_body(p_ref, u_ref, degp_ref, b2_ref,
              l1w1_ref, l1b1_ref, l1w2_ref, l1b2_ref,
              l2w1_ref, l2b1_ref, l2w2_ref, l2b2_ref,
              noise_ref, h_ref, z_ref):
    i = pl.program_id(0)
    dinv = _dinv_of(degp_ref)
    h = (p_ref[0] + p_ref[1] + u_ref[...]) * dinv[:, None] + b2_ref[...]
    h_ref[...] = h
    g1 = _gelu(jnp.dot(h, l1w1_ref[...],
                       preferred_element_type=jnp.float32) + l1b1_ref[...])
    mu = jnp.dot(g1, l1w2_ref[...],
                 preferred_element_type=jnp.float32) + l1b2_ref[...]
    g2 = _gelu(jnp.dot(h, l2w1_ref[...],
                       preferred_element_type=jnp.float32) + l2b1_ref[...])
    ls = jnp.dot(g2, l2w2_ref[...],
                 preferred_element_type=jnp.float32) + l2b2_ref[...]
    ls = jnp.minimum(ls, 10.0)
    z = mu + noise_ref[...] * jnp.exp(ls)
    rows = i * BN + lax.broadcasted_iota(jnp.int32, (BN, 1), 0)
    z = jnp.where(rows < N, z, 0.0)
    zs = jnp.sum(z, axis=0, keepdims=True)

    @pl.when(i == 0)
    def _():
        z_ref[...] = zs

    @pl.when(i > 0)
    def _():
        z_ref[...] = z_ref[...] + zs


def _tc_mm3(P2, u2, degp, b2, l1w1, l1b1, l1w2, l1b2,
            l2w1, l2b1, l2w2, l2b2, noise):
    return pl.pallas_call(
        _mm3_body,
        grid=(NP // BN,),
        in_specs=[
            pl.BlockSpec((NC, BN, H), lambda i: (0, i, 0)),
            pl.BlockSpec((BN, H), lambda i: (i, 0)),
            pl.BlockSpec((BN, NC), lambda i: (i, 0)),
            pl.BlockSpec((1, H), lambda i: (0, 0)),
            pl.BlockSpec((H, H), lambda i: (0, 0)),
            pl.BlockSpec((1, H), lambda i: (0, 0)),
            pl.BlockSpec((H, L), lambda i: (0, 0)),
            pl.BlockSpec((1, L), lambda i: (0, 0)),
            pl.BlockSpec((H, H), lambda i: (0, 0)),
            pl.BlockSpec((1, H), lambda i: (0, 0)),
            pl.BlockSpec((H, L), lambda i: (0, 0)),
            pl.BlockSpec((1, L), lambda i: (0, 0)),
            pl.BlockSpec((BN, L), lambda i: (i, 0)),
        ],
        out_specs=[
            pl.BlockSpec((BN, H), lambda i: (i, 0)),
            pl.BlockSpec((1, L), lambda i: (0, 0)),
        ],
        out_shape=[
            jax.ShapeDtypeStruct((NP, H), jnp.float32),
            jax.ShapeDtypeStruct((1, L), jnp.float32),
        ],
    )(P2, u2, degp, b2, l1w1, l1b1, l1w2, l1b2,
      l2w1, l2b1, l2w2, l2b2, noise)


def kernel(x, edge_index, W1, b1, W2, b2,
           l1w1, l1b1, l1w2, l1b2, l2w1, l2b1, l2w2, l2b2, noise):
    src = jnp.asarray(edge_index[0], jnp.int32)
    dst = jnp.asarray(edge_index[1], jnp.int32)
    pad_idx = N + (jnp.arange(EP - E, dtype=jnp.int32) % (NP - N))
    real_s = src.reshape(E // WK, WK)
    real_d = dst.reshape(E // WK, WK)
    pad2 = pad_idx.reshape((EP - E) // WK, WK)
    src2 = jnp.concatenate([real_s, pad2])[_WIN_PERM]
    dst2 = jnp.concatenate([real_d, pad2])[_WIN_PERM]
    zr1d = jnp.zeros((ROWS_T,), jnp.float32)
    zrows = jnp.zeros((ROWS_T, H), jnp.float32)
    xp = jnp.pad(x, ((0, NP - N), (0, 0)))
    noisep = jnp.pad(noise, ((0, NP - N), (0, 0)))

    degp = _sc_degrees(dst2, zr1d).reshape(NC, NP).T
    u1 = _tc_mm1(xp, W1, degp)
    P1 = _sc_edge_aggregate(u1, src2, dst2, zrows)
    u2 = _tc_mm2(P1, u1, degp, b1.reshape(1, H), W2)
    P2 = _sc_edge_aggregate(u2, src2, dst2, zrows)
    hp, z_global = _tc_mm3(
        P2, u2, degp, b2.reshape(1, H),
        l1w1, l1b1.reshape(1, H), l1w2, l1b2.reshape(1, L),
        l2w1, l2b1.reshape(1, H), l2w2, l2b2.reshape(1, L), noisep)
    return (z_global, hp[:N])


# speedup vs baseline: 1.0503x; 1.0503x over previous
import functools
import math

import jax
import jax.numpy as jnp
from jax import lax
from jax.experimental import pallas as pl
from jax.experimental.pallas import tpu as pltpu
from jax.experimental.pallas import tpu_sc as plsc

N = 10000
NP = 10240
E = 320000
D = 128
H = 128
L = 64

NC = 2
NS = 16
NW = NC * NS
WK = 128
WPER = 80
NWIN = NW * WPER
EP = NWIN * WK
ROWS_T = NP // NS

def _win_perm():
    import numpy as np
    perm = np.empty((NW, WPER), dtype=np.int32)
    r = 0
    d = E // WK
    for w in range(NW):
        nreal = 78 if w < 28 else 79
        perm[w, :nreal] = np.arange(r, r + nreal)
        perm[w, nreal:] = np.arange(d, d + WPER - nreal)
        r += nreal
        d += WPER - nreal
    return perm.reshape(-1)


_WIN_PERM = _win_perm()

_mesh = functools.partial(
    plsc.VectorSubcoreMesh,
    core_axis_name="c", subcore_axis_name="s",
    num_cores=NC, num_subcores=NS,
)


def _worker_id():
    c = lax.axis_index("c")
    s = lax.axis_index("s")
    return s * NC + c, c, s


def _deg_body(dst2_hbm, zr_hbm, out_hbm, dstb, ones_v, accd):
    wid, c, s = _worker_id()
    for i in range(WK // 16):
        ones_v[pl.ds(i * 16, 16)] = jnp.full((16,), 1.0, jnp.float32)
    pltpu.sync_copy(zr_hbm, accd.at[pl.ds(s * ROWS_T, ROWS_T)])
    pltpu.sync_copy(dst2_hbm.at[pl.ds(wid * WPER, WPER)], dstb)
    plsc.subcore_barrier()

    def body(j, carry):
        pltpu.sync_copy(ones_v, accd.at[dstb.at[j]], add=True)
        return carry

    lax.fori_loop(0, WPER, body, 0)

    plsc.subcore_barrier()
    pltpu.sync_copy(accd.at[pl.ds(s * ROWS_T, ROWS_T)],
                    out_hbm.at[pl.ds(c * NP + s * ROWS_T, ROWS_T)])


def _sc_degrees(dst2, zr1d):
    fn = pl.kernel(
        _deg_body,
        out_type=jax.ShapeDtypeStruct((NC * NP,), jnp.float32),
        mesh=_mesh(),
        scratch_types=[
            pltpu.VMEM((WPER, WK), jnp.int32),
            pltpu.VMEM((WK,), jnp.float32),
            pltpu.VMEM_SHARED((NP,), jnp.float32),
        ],
    )
    return fn(dst2, zr1d)


NBUF = 4
WSEG = 16


def _edge_body(u_hbm, src2_hbm, dst2_hbm, zr_hbm, out_hbm,
               srcb, dstb, rows, acc, g0, g1, g2, g3, s0, s1, s2, s3):
    wid, c, s = _worker_id()
    gsems = (g0, g1, g2, g3)
    ssems = (s0, s1, s2, s3)
    base = wid * WPER
    pltpu.sync_copy(zr_hbm, acc.at[pl.ds(s * ROWS_T, ROWS_T)])
    plsc.subcore_barrier()

    HK = WK // 2
    NLW = WSEG * 2

    def gstart(w, b):
        j, h = divmod(w, 2)
        return pltpu.async_copy(u_hbm.at[srcb.at[j, pl.ds(h * HK, HK)]],
                                rows.at[b], gsems[b])

    def sstart(w, b):
        j, h = divmod(w, 2)
        return pltpu.async_copy(rows.at[b],
                                acc.at[dstb.at[j, pl.ds(h * HK, HK)]],
                                ssems[b], add=True)

    for seg in range(WPER // WSEG):
        pltpu.sync_copy(src2_hbm.at[pl.ds(base + seg * WSEG, WSEG)], srcb)
        pltpu.sync_copy(dst2_hbm.at[pl.ds(base + seg * WSEG, WSEG)], dstb)
        gh = [gstart(0, 0), gstart(1, 1), None, None]
        sh = [None, None, None, None]
        for w in range(NLW):
            b = w % NBUF
            gh[b].wait()
            sh[b] = sstart(w, b)
            bp = (w + 2) % NBUF
            if sh[bp] is not None:
                sh[bp].wait()
            if w + 2 < NLW:
                gh[bp] = gstart(w + 2, bp)
        sh[(NLW - 2) % NBUF].wait()
        sh[(NLW - 1) % NBUF].wait()

    plsc.subcore_barrier()
    pltpu.sync_copy(acc.at[pl.ds(s * ROWS_T, ROWS_T)],
                    out_hbm.at[c, pl.ds(s * ROWS_T, ROWS_T)])


def _sc_edge_aggregate(u, src2, dst2, zrows):
    fn = pl.kernel(
        _edge_body,
        out_type=jax.ShapeDtypeStruct((NC, NP, H), jnp.float32),
        mesh=_mesh(),
        scratch_types=[
            pltpu.VMEM((WSEG, WK), jnp.int32),
            pltpu.VMEM((WSEG, WK), jnp.int32),
            pltpu.VMEM((NBUF, WK // 2, H), jnp.float32),
            pltpu.VMEM_SHARED((NP, H), jnp.float32),
            pltpu.SemaphoreType.DMA,
            pltpu.SemaphoreType.DMA,
            pltpu.SemaphoreType.DMA,
            pltpu.SemaphoreType.DMA,
            pltpu.SemaphoreType.DMA,
            pltpu.SemaphoreType.DMA,
            pltpu.SemaphoreType.DMA,
            pltpu.SemaphoreType.DMA,
        ],
        name="edge_aggregate",
    )
    return fn(u, src2, dst2, zrows)


BN = 2048


def _dinv_of(degp_ref):
    deg = degp_ref[:, 0] + degp_ref[:, 1] + 1.0
    return lax.rsqrt(deg)


def _mm1_body(x_ref, w_ref, degp_ref, o_ref):
    dinv = _dinv_of(degp_ref)
    hw = jnp.dot(x_ref[...], w_ref[...],
                 preferred_element_type=jnp.float32)
    o_ref[...] = hw * dinv[:, None]


def _tc_mm1(x, W1, degp):
    return pl.pallas_call(
        _mm1_body,
        grid=(NP // BN,),
        in_specs=[
            pl.BlockSpec((BN, D), lambda i: (i, 0)),
            pl.BlockSpec((D, H), lambda i: (0, 0)),
            pl.BlockSpec((BN, NC), lambda i: (i, 0)),
        ],
        out_specs=pl.BlockSpec((BN, H), lambda i: (i, 0)),
        out_shape=jax.ShapeDtypeStruct((NP, H), jnp.float32),
    )(x, W1, degp)


def _mm2_body(p_ref, u_ref, degp_ref, b1_ref, w2_ref, o_ref):
    dinv = _dinv_of(degp_ref)
    agg = (p_ref[0] + p_ref[1] + u_ref[...]) * dinv[:, None] + b1_ref[...]
    h1 = jnp.maximum(agg, 0.0)
    hw = jnp.dot(h1, w2_ref[...], preferred_element_type=jnp.float32)
    o_ref[...] = hw * dinv[:, None]


def _tc_mm2(P1, u1, degp, b1, W2):
    return pl.pallas_call(
        _mm2_body,
        grid=(NP // BN,),
        in_specs=[
            pl.BlockSpec((NC, BN, H), lambda i: (0, i, 0)),
            pl.BlockSpec((BN, H), lambda i: (i, 0)),
            pl.BlockSpec((BN, NC), lambda i: (i, 0)),
            pl.BlockSpec((1, H), lambda i: (0, 0)),
            pl.BlockSpec((H, H), lambda i: (0, 0)),
        ],
        out_specs=pl.BlockSpec((BN, H), lambda i: (i, 0)),
        out_shape=jax.ShapeDtypeStruct((NP, H), jnp.float32),
    )(P1, u1, degp, b1, W2)


_SQRT2 = math.sqrt(2.0)


def _gelu(x):
    return 0.5 * x * (1.0 + lax.erf(x / _SQRT2))


def _mm3_body(p_ref, u_ref, degp_ref, b2_ref,
              l1w1_ref, l1b1_ref, l1w2_ref, l1b2_ref,
              l2w1_ref, l2b1_ref, l2w2_ref, l2b2_ref,
              noise_ref, h_ref, z_ref):
    i = pl.program_id(0)
    dinv = _dinv_of(degp_ref)
    h = (p_ref[0] + p_ref[1] + u_ref[...]) * dinv[:, None] + b2_ref[...]
    h_ref[...] = h
    g1 = _gelu(jnp.dot(h, l1w1_ref[...],
                       preferred_element_type=jnp.float32) + l1b1_ref[...])
    mu = jnp.dot(g1, l1w2_ref[...],
                 preferred_element_type=jnp.float32) + l1b2_ref[...]
    g2 = _gelu(jnp.dot(h, l2w1_ref[...],
                       preferred_element_type=jnp.float32) + l2b1_ref[...])
    ls = jnp.dot(g2, l2w2_ref[...],
                 preferred_element_type=jnp.float32) + l2b2_ref[...]
    ls = jnp.minimum(ls, 10.0)
    z = mu + noise_ref[...] * jnp.exp(ls)
    rows = i * BN + lax.broadcasted_iota(jnp.int32, (BN, 1), 0)
    z = jnp.where(rows < N, z, 0.0)
    zs = jnp.sum(z, axis=0, keepdims=True)

    @pl.when(i == 0)
    def _():
        z_ref[...] = zs

    @pl.when(i > 0)
    def _():
        z_ref[...] = z_ref[...] + zs


def _tc_mm3(P2, u2, degp, b2, l1w1, l1b1, l1w2, l1b2,
            l2w1, l2b1, l2w2, l2b2, noise):
    return pl.pallas_call(
        _mm3_body,
        grid=(NP // BN,),
        in_specs=[
            pl.BlockSpec((NC, BN, H), lambda i: (0, i, 0)),
            pl.BlockSpec((BN, H), lambda i: (i, 0)),
            pl.BlockSpec((BN, NC), lambda i: (i, 0)),
            pl.BlockSpec((1, H), lambda i: (0, 0)),
            pl.BlockSpec((H, H), lambda i: (0, 0)),
            pl.BlockSpec((1, H), lambda i: (0, 0)),
            pl.BlockSpec((H, L), lambda i: (0, 0)),
            pl.BlockSpec((1, L), lambda i: (0, 0)),
            pl.BlockSpec((H, H), lambda i: (0, 0)),
            pl.BlockSpec((1, H), lambda i: (0, 0)),
            pl.BlockSpec((H, L), lambda i: (0, 0)),
            pl.BlockSpec((1, L), lambda i: (0, 0)),
            pl.BlockSpec((BN, L), lambda i: (i, 0)),
        ],
        out_specs=[
            pl.BlockSpec((BN, H), lambda i: (i, 0)),
            pl.BlockSpec((1, L), lambda i: (0, 0)),
        ],
        out_shape=[
            jax.ShapeDtypeStruct((NP, H), jnp.float32),
            jax.ShapeDtypeStruct((1, L), jnp.float32),
        ],
    )(P2, u2, degp, b2, l1w1, l1b1, l1w2, l1b2,
      l2w1, l2b1, l2w2, l2b2, noise)


def kernel(x, edge_index, W1, b1, W2, b2,
           l1w1, l1b1, l1w2, l1b2, l2w1, l2b1, l2w2, l2b2, noise):
    src = jnp.asarray(edge_index[0], jnp.int32)
    dst = jnp.asarray(edge_index[1], jnp.int32)
    pad_idx = N + (jnp.arange(EP - E, dtype=jnp.int32) % (NP - N))
    real_s = src.reshape(E // WK, WK)
    real_d = dst.reshape(E // WK, WK)
    pad2 = pad_idx.reshape((EP - E) // WK, WK)
    src2 = jnp.concatenate([real_s, pad2])[_WIN_PERM]
    dst2 = jnp.concatenate([real_d, pad2])[_WIN_PERM]
    zr1d = jnp.zeros((ROWS_T,), jnp.float32)
    zrows = jnp.zeros((ROWS_T, H), jnp.float32)
    xp = jnp.pad(x, ((0, NP - N), (0, 0)))
    noisep = jnp.pad(noise, ((0, NP - N), (0, 0)))

    degp = _sc_degrees(dst2, zr1d).reshape(NC, NP).T
    u1 = _tc_mm1(xp, W1, degp)
    P1 = _sc_edge_aggregate(u1, src2, dst2, zrows)
    u2 = _tc_mm2(P1, u1, degp, b1.reshape(1, H), W2)
    P2 = _sc_edge_aggregate(u2, src2, dst2, zrows)
    hp, z_global = _tc_mm3(
        P2, u2, degp, b2.reshape(1, H),
        l1w1, l1b1.reshape(1, H), l1w2, l1b2.reshape(1, L),
        l2w1, l2b1.reshape(1, H), l2w2, l2b2.reshape(1, L), noisep)
    return (z_global, hp[:N])


# speedup vs baseline: 1.1690x; 1.1130x over previous
import functools
import math

import jax
import jax.numpy as jnp
from jax import lax
from jax.experimental import pallas as pl
from jax.experimental.pallas import tpu as pltpu
from jax.experimental.pallas import tpu_sc as plsc

N = 10000
NP = 10240
E = 320000
D = 128
H = 128
L = 64

NC = 2
NS = 16
NW = NC * NS
WK = 128
WPER = 80
NWIN = NW * WPER
EP = NWIN * WK
ROWS_T = NP // NS

def _win_perm():
    import numpy as np
    perm = np.empty((NW, WPER), dtype=np.int32)
    r = 0
    d = E // WK
    for w in range(NW):
        nreal = 78 if w < 28 else 79
        perm[w, :nreal] = np.arange(r, r + nreal)
        perm[w, nreal:] = np.arange(d, d + WPER - nreal)
        r += nreal
        d += WPER - nreal
    return perm.reshape(-1)


_WIN_PERM = _win_perm()

_mesh = functools.partial(
    plsc.VectorSubcoreMesh,
    core_axis_name="c", subcore_axis_name="s",
    num_cores=NC, num_subcores=NS,
)


def _worker_id():
    c = lax.axis_index("c")
    s = lax.axis_index("s")
    return s * NC + c, c, s


def _deg_body(dst2_hbm, zr_hbm, out_hbm, dstb, ones_v, accd):
    wid, c, s = _worker_id()
    for i in range(WK // 16):
        ones_v[pl.ds(i * 16, 16)] = jnp.full((16,), 1.0, jnp.float32)
    pltpu.sync_copy(zr_hbm, accd.at[pl.ds(s * ROWS_T, ROWS_T)])
    pltpu.sync_copy(dst2_hbm.at[pl.ds(wid * WPER, WPER)], dstb)
    plsc.subcore_barrier()

    def body(j, carry):
        pltpu.sync_copy(ones_v, accd.at[dstb.at[j]], add=True)
        return carry

    lax.fori_loop(0, WPER, body, 0)

    plsc.subcore_barrier()
    pltpu.sync_copy(accd.at[pl.ds(s * ROWS_T, ROWS_T)],
                    out_hbm.at[pl.ds(c * NP + s * ROWS_T, ROWS_T)])


def _sc_degrees(dst2, zr1d):
    fn = pl.kernel(
        _deg_body,
        out_type=jax.ShapeDtypeStruct((NC * NP,), jnp.float32),
        mesh=_mesh(),
        scratch_types=[
            pltpu.VMEM((WPER, WK), jnp.int32),
            pltpu.VMEM((WK,), jnp.float32),
            pltpu.VMEM_SHARED((NP,), jnp.float32),
        ],
    )
    return fn(dst2, zr1d)


NBUF = 5
WSEG = 16


def _edge_body(u_hbm, src2_hbm, dst2_hbm, zr_hbm, out_hbm,
               srcb, dstb, rows, acc,
               g0, g1, g2, g3, g4, s0, s1, s2, s3, s4):
    wid, c, s = _worker_id()
    gsems = (g0, g1, g2, g3, g4)
    ssems = (s0, s1, s2, s3, s4)
    base = wid * WPER
    pltpu.sync_copy(zr_hbm, acc.at[pl.ds(s * ROWS_T, ROWS_T)])
    plsc.subcore_barrier()

    HK = WK // 2
    NLW = WSEG * 2

    def gstart(w, b):
        j, h = divmod(w, 2)
        return pltpu.async_copy(u_hbm.at[srcb.at[j, pl.ds(h * HK, HK)]],
                                rows.at[b], gsems[b])

    def sstart(w, b):
        j, h = divmod(w, 2)
        return pltpu.async_copy(rows.at[b],
                                acc.at[dstb.at[j, pl.ds(h * HK, HK)]],
                                ssems[b], add=True)

    for seg in range(WPER // WSEG):
        pltpu.sync_copy(src2_hbm.at[pl.ds(base + seg * WSEG, WSEG)], srcb)
        pltpu.sync_copy(dst2_hbm.at[pl.ds(base + seg * WSEG, WSEG)], dstb)
        gh = [gstart(0, 0), gstart(1, 1), gstart(2, 2), None, None]
        sh = [None] * NBUF
        for w in range(NLW):
            b = w % NBUF
            gh[b].wait()
            sh[b] = sstart(w, b)
            bp = (w + 3) % NBUF
            if sh[bp] is not None:
                sh[bp].wait()
            if w + 3 < NLW:
                gh[bp] = gstart(w + 3, bp)
        for t in range(2):
            sh[(NLW - 2 + t) % NBUF].wait()

    plsc.subcore_barrier()
    pltpu.sync_copy(acc.at[pl.ds(s * ROWS_T, ROWS_T)],
                    out_hbm.at[c, pl.ds(s * ROWS_T, ROWS_T)])


def _sc_edge_aggregate(u, src2, dst2, zrows):
    fn = pl.kernel(
        _edge_body,
        out_type=jax.ShapeDtypeStruct((NC, NP, H), jnp.float32),
        mesh=_mesh(),
        scratch_types=[
            pltpu.VMEM((WSEG, WK), jnp.int32),
            pltpu.VMEM((WSEG, WK), jnp.int32),
            pltpu.VMEM((NBUF, WK // 2, H), jnp.float32),
            pltpu.VMEM_SHARED((NP, H), jnp.float32),
        ] + [pltpu.SemaphoreType.DMA] * 10,
        name="edge_aggregate",
    )
    return fn(u, src2, dst2, zrows)


BN = 2048


def _dinv_of(degp_ref):
    deg = degp_ref[:, 0] + degp_ref[:, 1] + 1.0
    return lax.rsqrt(deg)


def _mm1_body(x_ref, w_ref, degp_ref, o_ref):
    dinv = _dinv_of(degp_ref)
    hw = jnp.dot(x_ref[...], w_ref[...],
                 preferred_element_type=jnp.float32)
    o_ref[...] = hw * dinv[:, None]


def _tc_mm1(x, W1, degp):
    return pl.pallas_call(
        _mm1_body,
        grid=(NP // BN,),
        in_specs=[
            pl.BlockSpec((BN, D), lambda i: (i, 0)),
            pl.BlockSpec((D, H), lambda i: (0, 0)),
            pl.BlockSpec((BN, NC), lambda i: (i, 0)),
        ],
        out_specs=pl.BlockSpec((BN, H), lambda i: (i, 0)),
        out_shape=jax.ShapeDtypeStruct((NP, H), jnp.float32),
    )(x, W1, degp)


def _mm2_body(p_ref, u_ref, degp_ref, b1_ref, w2_ref, o_ref):
    dinv = _dinv_of(degp_ref)
    agg = (p_ref[0] + p_ref[1] + u_ref[...]) * dinv[:, None] + b1_ref[...]
    h1 = jnp.maximum(agg, 0.0)
    hw = jnp.dot(h1, w2_ref[...], preferred_element_type=jnp.float32)
    o_ref[...] = hw * dinv[:, None]


def _tc_mm2(P1, u1, degp, b1, W2):
    return pl.pallas_call(
        _mm2_body,
        grid=(NP // BN,),
        in_specs=[
            pl.BlockSpec((NC, BN, H), lambda i: (0, i, 0)),
            pl.BlockSpec((BN, H), lambda i: (i, 0)),
            pl.BlockSpec((BN, NC), lambda i: (i, 0)),
            pl.BlockSpec((1, H), lambda i: (0, 0)),
            pl.BlockSpec((H, H), lambda i: (0, 0)),
        ],
        out_specs=pl.BlockSpec((BN, H), lambda i: (i, 0)),
        out_shape=jax.ShapeDtypeStruct((NP, H), jnp.float32),
    )(P1, u1, degp, b1, W2)


_SQRT2 = math.sqrt(2.0)


def _gelu(x):
    return 0.5 * x * (1.0 + lax.erf(x / _SQRT2))


def _mm3_body(p_ref, u_ref, degp_ref, b2_ref,
              l1w1_ref, l1b1_ref, l1w2_ref, l1b2_ref,
              l2w1_ref, l2b1_ref, l2w2_ref, l2b2_ref,
              noise_ref, h_ref, z_ref):
    i = pl.program_id(0)
    dinv = _dinv_of(degp_ref)
    h = (p_ref[0] + p_ref[1] + u_ref[...]) * dinv[:, None] + b2_ref[...]
    h_ref[...] = h
    g1 = _gelu(jnp.dot(h, l1w1_ref[...],
                       preferred_element_type=jnp.float32) + l1b1_ref[...])
    mu = jnp.dot(g1, l1w2_ref[...],
                 preferred_element_type=jnp.float32) + l1b2_ref[...]
    g2 = _gelu(jnp.dot(h, l2w1_ref[...],
                       preferred_element_type=jnp.float32) + l2b1_ref[...])
    ls = jnp.dot(g2, l2w2_ref[...],
                 preferred_element_type=jnp.float32) + l2b2_ref[...]
    ls = jnp.minimum(ls, 10.0)
    z = mu + noise_ref[...] * jnp.exp(ls)
    rows = i * BN + lax.broadcasted_iota(jnp.int32, (BN, 1), 0)
    z = jnp.where(rows < N, z, 0.0)
    zs = jnp.sum(z, axis=0, keepdims=True)

    @pl.when(i == 0)
    def _():
        z_ref[...] = zs

    @pl.when(i > 0)
    def _():
        z_ref[...] = z_ref[...] + zs


def _tc_mm3(P2, u2, degp, b2, l1w1, l1b1, l1w2, l1b2,
            l2w1, l2b1, l2w2, l2b2, noise):
    return pl.pallas_call(
        _mm3_body,
        grid=(NP // BN,),
        in_specs=[
            pl.BlockSpec((NC, BN, H), lambda i: (0, i, 0)),
            pl.BlockSpec((BN, H), lambda i: (i, 0)),
            pl.BlockSpec((BN, NC), lambda i: (i, 0)),
            pl.BlockSpec((1, H), lambda i: (0, 0)),
            pl.BlockSpec((H, H), lambda i: (0, 0)),
            pl.BlockSpec((1, H), lambda i: (0, 0)),
            pl.BlockSpec((H, L), lambda i: (0, 0)),
            pl.BlockSpec((1, L), lambda i: (0, 0)),
            pl.BlockSpec((H, H), lambda i: (0, 0)),
            pl.BlockSpec((1, H), lambda i: (0, 0)),
            pl.BlockSpec((H, L), lambda i: (0, 0)),
            pl.BlockSpec((1, L), lambda i: (0, 0)),
            pl.BlockSpec((BN, L), lambda i: (i, 0)),
        ],
        out_specs=[
            pl.BlockSpec((BN, H), lambda i: (i, 0)),
            pl.BlockSpec((1, L), lambda i: (0, 0)),
        ],
        out_shape=[
            jax.ShapeDtypeStruct((NP, H), jnp.float32),
            jax.ShapeDtypeStruct((1, L), jnp.float32),
        ],
    )(P2, u2, degp, b2, l1w1, l1b1, l1w2, l1b2,
      l2w1, l2b1, l2w2, l2b2, noise)


def kernel(x, edge_index, W1, b1, W2, b2,
           l1w1, l1b1, l1w2, l1b2, l2w1, l2b1, l2w2, l2b2, noise):
    src = jnp.asarray(edge_index[0], jnp.int32)
    dst = jnp.asarray(edge_index[1], jnp.int32)
    pad_idx = N + (jnp.arange(EP - E, dtype=jnp.int32) % (NP - N))
    real_s = src.reshape(E // WK, WK)
    real_d = dst.reshape(E // WK, WK)
    pad2 = pad_idx.reshape((EP - E) // WK, WK)

    src2 = jnp.concatenate([real_s, pad2])[_WIN_PERM]
    dst2 = jnp.concatenate([real_d, pad2])[_WIN_PERM]
    zr1d = jnp.zeros((ROWS_T,), jnp.float32)
    zrows = jnp.zeros((ROWS_T, H), jnp.float32)
    xp = jnp.pad(x, ((0, NP - N), (0, 0)))
    noisep = jnp.pad(noise, ((0, NP - N), (0, 0)))

    degp = _sc_degrees(dst2, zr1d).reshape(NC, NP).T
    u1 = _tc_mm1(xp, W1, degp)
    P1 = _sc_edge_aggregate(u1, src2, dst2, zrows)
    u2 = _tc_mm2(P1, u1, degp, b1.reshape(1, H), W2)
    P2 = _sc_edge_aggregate(u2, src2, dst2, zrows)
    hp, z_global = _tc_mm3(
        P2, u2, degp, b2.reshape(1, H),
        l1w1, l1b1.reshape(1, H), l1w2, l1b2.reshape(1, L),
        l2w1, l2b1.reshape(1, H), l2w2, l2b2.reshape(1, L), noisep)
    return (z_global, hp[:N])
